# baseline (speedup 1.0000x reference)
import jax
import jax.numpy as jnp
from jax.experimental import pallas as pl
from jax.experimental.pallas import tpu as pltpu

_MUS = (-0.9, -0.7, -0.5, -0.3, -0.1, 0.1, 0.3, 0.5, 0.7, 0.9, 1.0)
_SIGS = (0.1, 0.1, 0.1, 0.1, 0.1, 0.1, 0.1, 0.1, 0.1, 0.1, 0.001)
_PAD_ID = 0
_UNROLL = 8


def _issue_rows(doc_s, query_s, emb_h, de_buf, qe_buf, dsem, qsem):
    D = de_buf.shape[0]
    Q = qe_buf.shape[0]

    def outer(o, carry):
        base = o * _UNROLL
        for u in range(_UNROLL):
            tok = doc_s[0, 0, base + u]
            pltpu.make_async_copy(emb_h.at[pl.ds(tok, 1), :],
                                  de_buf.at[base + u], dsem).start()
        return carry
    jax.lax.fori_loop(0, D // _UNROLL, outer, 0)
    for i in range(Q):
        tok = query_s[0, 0, i]
        pltpu.make_async_copy(emb_h.at[pl.ds(tok, 1), :], qe_buf.at[i],
                              qsem).start()


def _compute(doc_v, query_c, w_s, b_s, out_ref, de_buf, qe_buf, de2d,
             dsem, qsem):
    Q = qe_buf.shape[0]
    pltpu.make_async_copy(qe_buf, qe_buf, qsem).wait()
    pltpu.make_async_copy(de_buf, de_buf, dsem).wait()

    de2d[...] = de_buf[:, 0, :]
    de = de2d[...]
    qe = qe_buf[:, 0, :]

    qsq = jnp.sum(qe * qe, axis=1, keepdims=True)
    qinv = 1.0 / (jnp.sqrt(qsq) + 1e-9)
    qmask = query_c[0] != _PAD_ID
    qn = qe * jnp.where(qmask, qinv, 0.0)

    dsq = de * de
    ones_r = jnp.ones((1, de.shape[1]), jnp.float32)
    nsq = jax.lax.dot_general(ones_r, dsq, (((1,), (1,)), ((), ())),
                              preferred_element_type=jnp.float32,
                              precision=jax.lax.Precision.HIGHEST)
    dmask = doc_v[0] != _PAD_ID
    dscale = jnp.where(dmask, 1.0 / (jnp.sqrt(nsq) + 1e-9), 0.0)

    sim_raw = jax.lax.dot_general(qn, de, (((1,), (1,)), ((), ())),
                                  preferred_element_type=jnp.float32,
                                  precision=jax.lax.Precision.HIGHEST)
    sim = sim_raw * dscale

    rowsum = jnp.sum(sim, axis=1, keepdims=True)
    rmask = rowsum != 0.0

    acc = jnp.zeros((Q, 1), jnp.float32)
    for k in range(11):
        c = -0.5 / (_SIGS[k] * _SIGS[k])
        t = sim - _MUS[k]
        sk = jnp.sum(jnp.exp(t * t * c), axis=1, keepdims=True)
        acc = acc + jnp.where(rmask, jnp.log(sk + 1e-6), 0.0) * w_s[k, 0]

    out_ref[0] = jnp.sum(acc, axis=0, keepdims=True) + b_s[0]


def _knrm_body(doc_cur, query_cur, doc_nxt, query_nxt, doc_v, query_c, w_s,
               b_s, emb_h, out_ref, de_a, qe_a, de_b, qe_b, de2d,
               dsem_a, qsem_a, dsem_b, qsem_b):
    j = pl.program_id(1)
    nsteps = pl.num_programs(1)

    @pl.when(j == 0)
    def _prologue():
        _issue_rows(doc_cur, query_cur, emb_h, de_a, qe_a, dsem_a, qsem_a)

    even = (j % 2) == 0
    last = j == nsteps - 1

    @pl.when(even)
    def _even_step():
        @pl.when(~last)
        def _():
            _issue_rows(doc_nxt, query_nxt, emb_h, de_b, qe_b, dsem_b, qsem_b)
        _compute(doc_v, query_c, w_s, b_s, out_ref, de_a, qe_a, de2d,
                 dsem_a, qsem_a)

    @pl.when(~even)
    def _odd_step():
        @pl.when(~last)
        def _():
            _issue_rows(doc_nxt, query_nxt, emb_h, de_a, qe_a, dsem_a, qsem_a)
        _compute(doc_v, query_c, w_s, b_s, out_ref, de_b, qe_b, de2d,
                 dsem_b, qsem_b)


def kernel(doc, query, query_idf, emb, W_combine, b_combine):
    del query_idf
    B, D = doc.shape
    Q = query.shape[1]
    doc32 = doc.astype(jnp.int32).reshape(B, 1, D)
    q32 = query.astype(jnp.int32).reshape(B, 1, Q)
    qcol = query.astype(jnp.int32).reshape(B, Q, 1)
    E = emb.shape[1]
    ncore = 2
    nstep = B // ncore

    def cur3(i, j):
        return (i * nstep + j, 0, 0)

    def nxt3(i, j):
        return (i * nstep + jnp.minimum(j + 1, nstep - 1), 0, 0)

    scores = pl.pallas_call(
        _knrm_body,
        grid=(ncore, nstep),
        in_specs=[
            pl.BlockSpec((1, 1, D), cur3, memory_space=pltpu.SMEM),
            pl.BlockSpec((1, 1, Q), cur3, memory_space=pltpu.SMEM),
            pl.BlockSpec((1, 1, D), nxt3, memory_space=pltpu.SMEM),
            pl.BlockSpec((1, 1, Q), nxt3, memory_space=pltpu.SMEM),
            pl.BlockSpec((1, 1, D), cur3),
            pl.BlockSpec((1, Q, 1), cur3),
            pl.BlockSpec((11, 1), lambda i, j: (0, 0), memory_space=pltpu.SMEM),
            pl.BlockSpec((1,), lambda i, j: (0,), memory_space=pltpu.SMEM),
            pl.BlockSpec(memory_space=pl.ANY),
        ],
        out_specs=pl.BlockSpec((1, 1, 1), cur3),
        out_shape=jax.ShapeDtypeStruct((B, 1, 1), jnp.float32),
        scratch_shapes=[
            pltpu.VMEM((D, 1, E), jnp.float32),
            pltpu.VMEM((Q, 1, E), jnp.float32),
            pltpu.VMEM((D, 1, E), jnp.float32),
            pltpu.VMEM((Q, 1, E), jnp.float32),
            pltpu.VMEM((D, E), jnp.float32),
            pltpu.SemaphoreType.DMA,
            pltpu.SemaphoreType.DMA,
            pltpu.SemaphoreType.DMA,
            pltpu.SemaphoreType.DMA,
        ],
        compiler_params=pltpu.CompilerParams(
            dimension_semantics=("parallel", "arbitrary"),
        ),
    )(doc32, q32, doc32, q32, doc32, qcol, W_combine, b_combine, emb)
    return scores.reshape(B, 1)


# speedup vs baseline: 1.2800x; 1.2800x over previous
import jax
import jax.numpy as jnp
from jax.experimental import pallas as pl
from jax.experimental.pallas import tpu as pltpu

_MUS = (-0.9, -0.7, -0.5, -0.3, -0.1, 0.1, 0.3, 0.5, 0.7, 0.9, 1.0)
_SIGS = (0.1, 0.1, 0.1, 0.1, 0.1, 0.1, 0.1, 0.1, 0.1, 0.1, 0.001)
_PAD_ID = 0
_UNROLL = 8


def _issue_rows(doc_s, query_s, emb_h, de_buf, qe_buf, dsem, qsem):
    D = de_buf.shape[0]
    Q = qe_buf.shape[0]

    def outer(o, carry):
        base = o * _UNROLL
        for u in range(_UNROLL):
            tok = doc_s[0, 0, base + u]
            pltpu.make_async_copy(emb_h.at[pl.ds(tok, 1), :],
                                  de_buf.at[base + u], dsem).start()
        return carry
    jax.lax.fori_loop(0, D // _UNROLL, outer, 0)
    for i in range(Q):
        tok = query_s[0, 0, i]
        pltpu.make_async_copy(emb_h.at[pl.ds(tok, 1), :], qe_buf.at[i],
                              qsem).start()


def _compute(doc_v, query_c, w_s, b_s, out_ref, de_buf, qe_buf, de2d,
             dsem, qsem):
    Q = qe_buf.shape[0]
    pltpu.make_async_copy(qe_buf, qe_buf, qsem).wait()
    pltpu.make_async_copy(de_buf, de_buf, dsem).wait()

    de2d[...] = de_buf[:, 0, :]
    de = de2d[...]
    qe = qe_buf[:, 0, :]

    qsq = jnp.sum(qe * qe, axis=1, keepdims=True)
    qinv = 1.0 / (jnp.sqrt(qsq) + 1e-9)
    qmask = query_c[0] != _PAD_ID
    qn = qe * jnp.where(qmask, qinv, 0.0)

    dsq = de * de
    ones_c = jnp.ones((1, de.shape[1]), jnp.float32)
    nsq = jax.lax.dot_general(dsq, ones_c, (((1,), (1,)), ((), ())),
                              preferred_element_type=jnp.float32,
                              precision=jax.lax.Precision.HIGHEST)
    dn = de * (1.0 / (jnp.sqrt(nsq) + 1e-9))

    sim_raw = jax.lax.dot_general(
        qn.astype(jnp.bfloat16), dn.astype(jnp.bfloat16),
        (((1,), (1,)), ((), ())),
        preferred_element_type=jnp.float32)
    dmask = doc_v[0] != _PAD_ID
    sim = jnp.where(dmask, sim_raw, 0.0)

    rowsum = jnp.sum(sim, axis=1, keepdims=True)
    rmask = rowsum != 0.0

    scorev = jnp.zeros((1, 1), jnp.float32)
    for k in range(11):
        c = -0.5 / (_SIGS[k] * _SIGS[k])
        t = sim - _MUS[k]
        sk = jnp.sum(jnp.exp(t * t * c), axis=1, keepdims=True)
        lg = jnp.where(rmask, jnp.log(sk + 1e-6), 0.0)
        phi = jnp.sum(lg, axis=0, keepdims=True)
        pb = phi.astype(jnp.bfloat16).astype(jnp.float32)
        wb = jnp.full((1, 1), w_s[k, 0], jnp.float32)
        wb = wb.astype(jnp.bfloat16).astype(jnp.float32)
        scorev = scorev + pb * wb

    out_ref[0] = scorev + b_s[0]


def _knrm_body(doc_cur, query_cur, doc_nxt, query_nxt, doc_v, query_c, w_s,
               b_s, emb_h, out_ref, de_a, qe_a, de_b, qe_b, de2d,
               dsem_a, qsem_a, dsem_b, qsem_b):
    j = pl.program_id(1)
    nsteps = pl.num_programs(1)

    @pl.when(j == 0)
    def _prologue():
        _issue_rows(doc_cur, query_cur, emb_h, de_a, qe_a, dsem_a, qsem_a)

    even = (j % 2) == 0
    last = j == nsteps - 1

    @pl.when(even)
    def _even_step():
        @pl.when(~last)
        def _():
            _issue_rows(doc_nxt, query_nxt, emb_h, de_b, qe_b, dsem_b, qsem_b)
        _compute(doc_v, query_c, w_s, b_s, out_ref, de_a, qe_a, de2d,
                 dsem_a, qsem_a)

    @pl.when(~even)
    def _odd_step():
        @pl.when(~last)
        def _():
            _issue_rows(doc_nxt, query_nxt, emb_h, de_a, qe_a, dsem_a, qsem_a)
        _compute(doc_v, query_c, w_s, b_s, out_ref, de_b, qe_b, de2d,
                 dsem_b, qsem_b)


def kernel(doc, query, query_idf, emb, W_combine, b_combine):
    del query_idf
    B, D = doc.shape
    Q = query.shape[1]
    doc32 = doc.astype(jnp.int32).reshape(B, 1, D)
    q32 = query.astype(jnp.int32).reshape(B, 1, Q)
    qcol = query.astype(jnp.int32).reshape(B, Q, 1)
    E = emb.shape[1]
    ncore = 2
    nstep = B // ncore

    def cur3(i, j):
        return (i * nstep + j, 0, 0)

    def nxt3(i, j):
        return (i * nstep + jnp.minimum(j + 1, nstep - 1), 0, 0)

    scores = pl.pallas_call(
        _knrm_body,
        grid=(ncore, nstep),
        in_specs=[
            pl.BlockSpec((1, 1, D), cur3, memory_space=pltpu.SMEM),
            pl.BlockSpec((1, 1, Q), cur3, memory_space=pltpu.SMEM),
            pl.BlockSpec((1, 1, D), nxt3, memory_space=pltpu.SMEM),
            pl.BlockSpec((1, 1, Q), nxt3, memory_space=pltpu.SMEM),
            pl.BlockSpec((1, 1, D), cur3),
            pl.BlockSpec((1, Q, 1), cur3),
            pl.BlockSpec((11, 1), lambda i, j: (0, 0), memory_space=pltpu.SMEM),
            pl.BlockSpec((1,), lambda i, j: (0,), memory_space=pltpu.SMEM),
            pl.BlockSpec(memory_space=pl.ANY),
        ],
        out_specs=pl.BlockSpec((1, 1, 1), cur3),
        out_shape=jax.ShapeDtypeStruct((B, 1, 1), jnp.float32),
        scratch_shapes=[
            pltpu.VMEM((D, 1, E), jnp.float32),
            pltpu.VMEM((Q, 1, E), jnp.float32),
            pltpu.VMEM((D, 1, E), jnp.float32),
            pltpu.VMEM((Q, 1, E), jnp.float32),
            pltpu.VMEM((D, E), jnp.float32),
            pltpu.SemaphoreType.DMA,
            pltpu.SemaphoreType.DMA,
            pltpu.SemaphoreType.DMA,
            pltpu.SemaphoreType.DMA,
        ],
        compiler_params=pltpu.CompilerParams(
            dimension_semantics=("parallel", "arbitrary"),
        ),
    )(doc32, q32, doc32, q32, doc32, qcol, W_combine, b_combine, emb)
    return scores.reshape(B, 1)
